# baseline (speedup 1.0000x reference)
import jax
import jax.numpy as jnp
from jax import lax
from jax.experimental import pallas as pl
from jax.experimental.pallas import tpu as pltpu
from jax.experimental.pallas import tpu_sc as plsc

VOCAB = 100000
EMBED = 64
PADW = 128
MAX_LEN = 200
BATCH = 4096

NUM_CORES = 2
NUM_SUBCORES = 16
NUM_WORKERS = NUM_CORES * NUM_SUBCORES
SEQ_PER_W = BATCH // NUM_WORKERS
ROWS = MAX_LEN
N_CHUNKS = SEQ_PER_W
N_PAIRS = N_CHUNKS // 2
LANES = 16
COLS = EMBED // LANES


def _body(seq_hbm, table_hbm, pe_hbm, out_hbm, pe_v,
          x0, x1, g0, g1, r0, r1,
          xs0, xs1, gs0, gs1, os0, os1):
    idxb = (x0, x1)
    gbuf = (g0, g1)
    rbuf = (r0, r1)
    xsem = (xs0, xs1)
    gsem = (gs0, gs1)
    osem = (os0, os1)
    wid = lax.axis_index("s") * NUM_CORES + lax.axis_index("c")
    base = wid * SEQ_PER_W * MAX_LEN

    pltpu.sync_copy(pe_hbm, pe_v)

    def idx_start(i, b):
        src = seq_hbm.at[pl.ds(base + i * ROWS, ROWS)]
        pltpu.make_async_copy(src, idxb[b], xsem[b]).start()

    def idx_wait(i, b):
        src = seq_hbm.at[pl.ds(base + i * ROWS, ROWS)]
        pltpu.make_async_copy(src, idxb[b], xsem[b]).wait()

    def gather_start(b):
        pltpu.make_async_copy(table_hbm.at[idxb[b]], gbuf[b], gsem[b]).start()

    def gather_wait(b):
        pltpu.make_async_copy(table_hbm.at[idxb[b]], gbuf[b], gsem[b]).wait()

    def out_start(i, b):
        dst = out_hbm.at[pl.ds(base + i * ROWS, ROWS)]
        pltpu.make_async_copy(rbuf[b], dst, osem[b]).start()

    def out_wait(i, b):
        dst = out_hbm.at[pl.ds(base + i * ROWS, ROWS)]
        pltpu.make_async_copy(rbuf[b], dst, osem[b]).wait()

    def add_pe(b):
        def add_row(r, _):
            for c in range(COLS):
                rbuf[b][r, pl.ds(c * LANES, LANES)] = (
                    gbuf[b][r, pl.ds(c * LANES, LANES)]
                    + pe_v[pl.ds(r * EMBED + c * LANES, LANES)]
                )
            return 0
        lax.fori_loop(0, MAX_LEN, add_row, 0, unroll=8)

    idx_start(0, 0)
    idx_start(1, 1)
    idx_wait(0, 0)
    gather_start(0)

    def pair(p, _):
        for j in range(2):
            i = p * 2 + j
            b = j
            nb = 1 - j
            if j == 0:
                @pl.when(p < N_PAIRS - 1)
                def _():
                    idx_wait(i + 1, nb)
                    gather_start(nb)
            else:
                idx_wait(i + 1, nb)
                gather_start(nb)
            gather_wait(b)
            if j == 0:
                idx_start(i + 2, b)
            else:
                @pl.when(p < N_PAIRS - 1)
                def _():
                    idx_start(i + 2, b)
            if j == 0:
                @pl.when(p > 0)
                def _():
                    out_wait(i - 2, b)
            else:
                @pl.when(p > 0)
                def _():
                    out_wait(i - 2, b)
            add_pe(b)
            out_start(i, b)
        return 0

    lax.fori_loop(0, N_PAIRS - 1, pair, 0)

    i = N_CHUNKS - 2
    idx_wait(i + 1, 1)
    gather_start(1)
    gather_wait(0)
    out_wait(i - 2, 0)
    add_pe(0)
    out_start(i, 0)
    gather_wait(1)
    out_wait(i - 1, 1)
    add_pe(1)
    out_start(i + 1, 1)
    out_wait(i, 0)
    out_wait(i + 1, 1)


@jax.jit
def _run(seq_flat, table_pad, pe_flat):
    mesh = plsc.VectorSubcoreMesh(core_axis_name="c", subcore_axis_name="s")
    return pl.kernel(
        _body,
        out_type=jax.ShapeDtypeStruct((BATCH * MAX_LEN, EMBED), jnp.float32),
        mesh=mesh,
        compiler_params=pltpu.CompilerParams(use_tc_tiling_on_sc=True),
        scratch_types=[
            pltpu.VMEM((MAX_LEN * EMBED,), jnp.float32),
            pltpu.VMEM((ROWS,), jnp.int32),
            pltpu.VMEM((ROWS,), jnp.int32),
            pltpu.VMEM((ROWS, PADW), jnp.float32),
            pltpu.VMEM((ROWS, PADW), jnp.float32),
            pltpu.VMEM((ROWS, EMBED), jnp.float32),
            pltpu.VMEM((ROWS, EMBED), jnp.float32),
            pltpu.SemaphoreType.DMA,
            pltpu.SemaphoreType.DMA,
            pltpu.SemaphoreType.DMA,
            pltpu.SemaphoreType.DMA,
            pltpu.SemaphoreType.DMA,
            pltpu.SemaphoreType.DMA,
        ],
    )(seq_flat, table_pad, pe_flat)


def kernel(sequence, token_table, pe_weight):
    seq_flat = sequence.reshape(-1).astype(jnp.int32)
    table_pad = jnp.pad(token_table, ((0, 0), (0, PADW - EMBED)))
    pe_flat = pe_weight.reshape(-1)
    out = _run(seq_flat, table_pad, pe_flat)
    return out.reshape(BATCH, MAX_LEN, EMBED)
